# Initial kernel scaffold; baseline (speedup 1.0000x reference)
import jax
import jax.numpy as jnp
from jax.experimental import pallas as pl


def kernel(x, edge_index, adj_vals, W0, b0, W1, b1, g1, g2):
    raise NotImplementedError("write your pallas kernel here")


# speedup vs baseline: 11.0621x; 11.0621x over previous
import functools

import jax
import jax.numpy as jnp
from jax import lax
from jax.experimental import pallas as pl
from jax.experimental.pallas import tpu as pltpu
from jax.experimental.pallas import tpu_sc as plsc

N = 10000
NPAD = 10240
H = 128
C = 64
EPS = 0.1
NC = 2
NS = 16
NT = NC * NS
CHUNK = 128
LANES = 16
RPT = NPAD // NS
ZROWS = 128


def _tc1_body(x_ref, w0_ref, b0_ref, g_ref, h_ref, x12_ref):
    h = jnp.dot(x_ref[...], w0_ref[...], preferred_element_type=jnp.float32)
    h = jnp.maximum(h + b0_ref[...], 0.0)
    h_ref[...] = h
    x12_ref[...] = lax.dot_general(
        g_ref[...], h, (((1,), (1,)), ((), ())),
        preferred_element_type=jnp.float32)


def _tc2_body(r_ref, h0_ref, g_ref, h_ref, x12_ref):
    hn = EPS * h0_ref[...] + r_ref[0, :N] + r_ref[1, :N]
    h_ref[...] = hn
    x12_ref[...] = lax.dot_general(
        g_ref[...], hn, (((1,), (1,)), ((), ())),
        preferred_element_type=jnp.float32)


def _tc3_body(r_ref, h0_ref, w1_ref, b1_ref, o_ref):
    hn = EPS * h0_ref[...] + r_ref[0, :N] + r_ref[1, :N]
    o_ref[...] = jnp.dot(hn, w1_ref[...],
                         preferred_element_type=jnp.float32) + b1_ref[...]


_tc1 = pl.pallas_call(
    _tc1_body,
    out_shape=[jax.ShapeDtypeStruct((N, H), jnp.float32),
               jax.ShapeDtypeStruct((8, N), jnp.float32)],
)

_tc2 = pl.pallas_call(
    _tc2_body,
    out_shape=[jax.ShapeDtypeStruct((N, H), jnp.float32),
               jax.ShapeDtypeStruct((8, N), jnp.float32)],
)

_tc3 = pl.pallas_call(
    _tc3_body,
    out_shape=jax.ShapeDtypeStruct((N, C), jnp.float32),
)


def _sc_edge_body(nchunk, src_h, dst_h, adj_h, x12_h, h_h, out_h,
                  x1_v, x2_v, sidx, didx, adj_v, m_v, rows_v, res_sh, sem):
    c = lax.axis_index("c")
    s = lax.axis_index("s")
    wid = c * NS + s
    ept = nchunk * CHUNK

    pltpu.sync_copy(x12_h.at[0], x1_v)
    pltpu.sync_copy(x12_h.at[1], x2_v)

    zero16 = jnp.zeros((LANES,), jnp.float32)

    def _zrow(i, carry):
        for g in range(H // LANES):
            rows_v[i, pl.ds(g * LANES, LANES)] = zero16
        return carry

    lax.fori_loop(0, CHUNK, _zrow, 0)
    for k in range(RPT // ZROWS):
        pltpu.sync_copy(rows_v.at[pl.ds(0, ZROWS)],
                        res_sh.at[pl.ds(s * RPT + k * ZROWS, ZROWS)])
    plsc.subcore_barrier()

    def _chunk(ci, carry):
        base = wid * ept + ci * CHUNK
        pltpu.sync_copy(src_h.at[pl.ds(base, CHUNK)], sidx)
        pltpu.sync_copy(dst_h.at[pl.ds(base, CHUNK)], didx)
        pltpu.sync_copy(adj_h.at[pl.ds(base, CHUNK)], adj_v)
        pltpu.async_copy(h_h.at[didx], rows_v, sem).wait()
        for g in range(CHUNK // LANES):
            sv = sidx[pl.ds(g * LANES, LANES)]
            dv = didx[pl.ds(g * LANES, LANES)]
            av = adj_v[pl.ds(g * LANES, LANES)]
            z = plsc.load_gather(x1_v, [sv]) + plsc.load_gather(x2_v, [dv])
            az = jnp.abs(z)
            e = jnp.exp(az * (-2.0))
            t = (1.0 - e) / (1.0 + e)
            m_v[pl.ds(g * LANES, LANES)] = jnp.sign(z) * t * av
        def _scale(ei, carry):
            mb = plsc.load_gather(m_v, [jnp.zeros((LANES,), jnp.int32) + ei])
            for g in range(H // LANES):
                rows_v[ei, pl.ds(g * LANES, LANES)] = (
                    rows_v[ei, pl.ds(g * LANES, LANES)] * mb)
            return carry

        lax.fori_loop(0, CHUNK, _scale, 0)
        pltpu.sync_copy(rows_v, res_sh.at[sidx], add=True)
        return carry

    lax.fori_loop(0, nchunk, _chunk, 0)
    plsc.subcore_barrier()

    for k in range(RPT // ZROWS):
        r0 = s * RPT + k * ZROWS
        pltpu.sync_copy(res_sh.at[pl.ds(r0, ZROWS)],
                        out_h.at[c, pl.ds(r0, ZROWS)])


@functools.cache
def _make_sc_kernel(nchunk):
    mesh = plsc.VectorSubcoreMesh(core_axis_name="c", subcore_axis_name="s",
                                  num_cores=NC, num_subcores=NS)
    return pl.kernel(
        functools.partial(_sc_edge_body, nchunk),
        out_type=jax.ShapeDtypeStruct((NC, NPAD, H), jnp.float32),
        mesh=mesh,
        compiler_params=pltpu.CompilerParams(needs_layout_passes=False),
        scratch_types=[
            pltpu.VMEM((N,), jnp.float32),
            pltpu.VMEM((N,), jnp.float32),
            pltpu.VMEM((CHUNK,), jnp.int32),
            pltpu.VMEM((CHUNK,), jnp.int32),
            pltpu.VMEM((CHUNK,), jnp.float32),
            pltpu.VMEM((CHUNK,), jnp.float32),
            pltpu.VMEM((CHUNK, H), jnp.float32),
            pltpu.VMEM_SHARED((NPAD, H), jnp.float32),
            pltpu.SemaphoreType.DMA,
        ],
    )


def kernel(x, edge_index, adj_vals, W0, b0, W1, b1, g1, g2):
    src = edge_index[0].astype(jnp.int32)
    dst = edge_index[1].astype(jnp.int32)
    e_total = src.shape[0]
    nchunk = -(-e_total // (NT * CHUNK))
    epad = nchunk * CHUNK * NT
    pad = epad - e_total
    if pad:
        src = jnp.concatenate([src, jnp.zeros((pad,), jnp.int32)])
        dst = jnp.concatenate([dst, jnp.zeros((pad,), jnp.int32)])
        adj = jnp.concatenate([adj_vals, jnp.zeros((pad,), jnp.float32)])
    else:
        adj = adj_vals

    zpad = jnp.zeros((6, H), jnp.float32)
    g_a = jnp.concatenate([g1[0:1], g2[0:1], zpad])
    g_b = jnp.concatenate([g1[1:2], g2[1:2], zpad])

    sc_k = _make_sc_kernel(nchunk)

    h0, x12 = _tc1(x, W0, b0[None, :], g_a)
    res = sc_k(src, dst, adj, x12, h0)
    h1, x12 = _tc2(res, h0, g_b)
    res = sc_k(src, dst, adj, x12, h1)
    return _tc3(res, h0, W1, b1[None, :])
